# baseline (speedup 1.0000x reference)
import functools

import jax
import jax.numpy as jnp
from jax import lax
from jax.experimental import pallas as pl
from jax.experimental.pallas import tpu as pltpu
from jax.experimental.pallas import tpu_sc as plsc

VOCAB = 1000000
EMBED_DIM = 64
OUTPUT_DIM = 2
BATCH = 4096
SEQ = 200

_INFO = plsc.get_sparse_core_info()
_NC = _INFO.num_cores
_NS = _INFO.num_subcores
_NW = _NC * _NS
_B_PER_W = BATCH // _NW
_E = 4
_CHUNK_ROWS = _E * SEQ
_N_CHUNKS = _B_PER_W // _E


def _sc_pooled_sums(ids_flat, table):
    mesh = plsc.VectorSubcoreMesh(core_axis_name="c", subcore_axis_name="s")

    @functools.partial(
        pl.kernel,
        mesh=mesh,
        out_type=jax.ShapeDtypeStruct((BATCH, EMBED_DIM), jnp.float32),
        scratch_types=[
            pltpu.VMEM((_B_PER_W * SEQ,), jnp.int32),
            pltpu.VMEM((_CHUNK_ROWS, EMBED_DIM), jnp.float32),
            pltpu.VMEM((_B_PER_W, EMBED_DIM), jnp.float32),
            pltpu.SemaphoreType.DMA,
        ],
        compiler_params=pltpu.CompilerParams(use_tc_tiling_on_sc=False),
    )
    def k(ids_hbm, table_hbm, out_hbm, idx_v, rows_v, acc_v, sem):
        wid = lax.axis_index("s") * _NC + lax.axis_index("c")
        base = wid * (_B_PER_W * SEQ)
        pltpu.sync_copy(ids_hbm.at[pl.ds(base, _B_PER_W * SEQ)], idx_v)

        def chunk_body(c, carry_unused):
            pltpu.async_copy(
                table_hbm.at[idx_v.at[pl.ds(c * _CHUNK_ROWS, _CHUNK_ROWS)]],
                rows_v,
                sem,
            ).wait()
            for e in range(_E):
                def row_body(r, carry):
                    parts = []
                    for d in range(EMBED_DIM // 16):
                        parts.append(
                            carry[d] + rows_v[e * SEQ + r, pl.ds(d * 16, 16)]
                        )
                    return tuple(parts)

                zeros = tuple(
                    jnp.zeros((16,), jnp.float32)
                    for _ in range(EMBED_DIM // 16)
                )
                acc = lax.fori_loop(0, SEQ, row_body, zeros)
                for d in range(EMBED_DIM // 16):
                    acc_v[c * _E + e, pl.ds(d * 16, 16)] = acc[d]
            return carry_unused

        lax.fori_loop(0, _N_CHUNKS, chunk_body, 0)
        pltpu.sync_copy(acc_v, out_hbm.at[pl.ds(wid * _B_PER_W, _B_PER_W)])

    return k(ids_flat, table)


def _tc_linear(sums, W, b):
    def body(s_ref, w_ref, b_ref, o_ref):
        pooled = s_ref[...] * (1.0 / SEQ)
        o_ref[...] = (
            jnp.dot(pooled, w_ref[...], preferred_element_type=jnp.float32)
            + b_ref[...]
        )

    return pl.pallas_call(
        body,
        out_shape=jax.ShapeDtypeStruct((BATCH, OUTPUT_DIM), jnp.float32),
    )(sums, W, b.reshape(1, OUTPUT_DIM))


@jax.jit
def kernel(ids, table, W, b):
    ids_flat = jnp.reshape(ids.astype(jnp.int32), (BATCH * SEQ,))
    sums = _sc_pooled_sums(ids_flat, table)
    return _tc_linear(sums, W, b)


# speedup vs baseline: 1.1467x; 1.1467x over previous
import functools

import jax
import jax.numpy as jnp
from jax import lax
from jax.experimental import pallas as pl
from jax.experimental.pallas import tpu as pltpu
from jax.experimental.pallas import tpu_sc as plsc

VOCAB = 1000000
EMBED_DIM = 64
OUTPUT_DIM = 2
BATCH = 4096
SEQ = 200

_INFO = plsc.get_sparse_core_info()
_NC = _INFO.num_cores
_NS = _INFO.num_subcores
_NW = _NC * _NS
_B_PER_W = BATCH // _NW
_K = 8
_D16 = EMBED_DIM // 16


def _sc_pooled_sums(ids_t, table):
    mesh = plsc.VectorSubcoreMesh(core_axis_name="c", subcore_axis_name="s")

    @functools.partial(
        pl.kernel,
        mesh=mesh,
        out_type=jax.ShapeDtypeStruct((BATCH, EMBED_DIM), jnp.float32),
        scratch_types=[
            pltpu.VMEM((SEQ, _B_PER_W), jnp.int32),
            pltpu.VMEM((_B_PER_W, EMBED_DIM), jnp.float32),
            pltpu.SemaphoreType.DMA,
        ],
        compiler_params=pltpu.CompilerParams(use_tc_tiling_on_sc=False),
    )
    def k(ids_hbm, table_hbm, out_hbm, idx_v, acc_v, sem):
        wid = lax.axis_index("s") * _NC + lax.axis_index("c")
        pltpu.sync_copy(ids_hbm.at[:, pl.ds(wid * _B_PER_W, _B_PER_W)], idx_v)

        zero = jnp.zeros((16,), jnp.float32)

        def zero_body(i, carry):
            for d in range(_D16):
                acc_v[i, pl.ds(d * 16, 16)] = zero
            return carry

        lax.fori_loop(0, _B_PER_W, zero_body, 0)

        def fire(r):
            return pltpu.async_copy(
                table_hbm.at[idx_v.at[r]], acc_v, sem, add=True
            )

        for j in range(_K):
            fire(j)

        def chunk_body(i, carry):
            for j in range(_K):
                fire(i * _K + j)
            for j in range(_K):
                pltpu.make_async_copy(table_hbm.at[idx_v.at[0]], acc_v, sem).wait()
            return carry

        lax.fori_loop(1, SEQ // _K, chunk_body, 0)
        for j in range(_K):
            pltpu.make_async_copy(table_hbm.at[idx_v.at[0]], acc_v, sem).wait()

        pltpu.sync_copy(acc_v, out_hbm.at[pl.ds(wid * _B_PER_W, _B_PER_W)])

    return k(ids_t, table)


def _tc_linear(sums, W, b):
    def body(s_ref, w_ref, b_ref, o_ref):
        pooled = s_ref[...] * (1.0 / SEQ)
        o_ref[...] = (
            jnp.dot(pooled, w_ref[...], preferred_element_type=jnp.float32)
            + b_ref[...]
        )

    return pl.pallas_call(
        body,
        out_shape=jax.ShapeDtypeStruct((BATCH, OUTPUT_DIM), jnp.float32),
    )(sums, W, b.reshape(1, OUTPUT_DIM))


@jax.jit
def kernel(ids, table, W, b):
    ids_t = jnp.transpose(ids.astype(jnp.int32))
    sums = _sc_pooled_sums(ids_t, table)
    return _tc_linear(sums, W, b)
